# baseline (speedup 1.0000x reference)
import functools

import jax
import jax.numpy as jnp
from jax import lax
from jax.experimental import pallas as pl
from jax.experimental.pallas import tpu as pltpu
from jax.experimental.pallas import tpu_sc as plsc

N = 10000
E = 320000
F = 128

NS = 16
L = 16
NP = 10240
ROWS_W = NP // NS
E_W = E // NS
CH = 16
NCHUNK = E_W // CH
NBUF = 5
NB_N = 3
ZR = 40
RBLK = 1000

_sc_mesh = plsc.VectorSubcoreMesh(
    core_axis_name="c", subcore_axis_name="s", num_cores=1)
_sc_params = pltpu.CompilerParams(needs_layout_passes=False)


def _rsqrt16(x):
    i = plsc.bitcast(x, jnp.int32)
    y = plsc.bitcast(jnp.int32(0x5F3759DF) - (i >> 1), jnp.float32)
    half = x * 0.5
    for _ in range(3):
        y = y * (1.5 - half * y * y)
    return y


@functools.partial(
    pl.kernel,
    out_type=(
        jax.ShapeDtypeStruct((N, F), jnp.float32),
        jax.ShapeDtypeStruct((NP,), jnp.float32),
    ),
    mesh=_sc_mesh,
    compiler_params=_sc_params,
    scratch_types=[
        pltpu.VMEM_SHARED((NS, 2, NP), jnp.float32),
        pltpu.VMEM((E_W,), jnp.int32),
        pltpu.VMEM((NP,), jnp.float32),
        pltpu.VMEM((NP,), jnp.float32),
        pltpu.VMEM((NS, 2, ROWS_W), jnp.float32),
        pltpu.VMEM((2, ROWS_W), jnp.float32),
        pltpu.VMEM((ROWS_W,), jnp.float32),
        pltpu.VMEM((NB_N, CH, F), jnp.float32),
        [pltpu.SemaphoreType.DMA] * NB_N,
        [pltpu.SemaphoreType.DMA] * NB_N,
    ],
)
def _norm_kernel(src_hbm, dst_hbm, fw_hbm, hw_hbm, nd_hbm, hist_sh, eidx_v, hist_o,
                 hist_i, tmp_v, deg_v, ns_v, row_v, fsems, wsems):
    sid = lax.axis_index("s")
    base = sid * E_W
    rbase = sid * ROWS_W

    zeros16 = jnp.zeros((L,), jnp.float32)

    def _zero(i, carry):
        hist_o[pl.ds(i * L, L)] = zeros16
        hist_i[pl.ds(i * L, L)] = zeros16
        return carry

    lax.fori_loop(0, NP // L, _zero, 0)

    ones16 = jnp.ones((L,), jnp.float32)

    def _accumulate(hist, ehbm):
        pltpu.sync_copy(ehbm.at[pl.ds(base, E_W)], eidx_v)

        def _body(j, carry):
            iv = eidx_v[pl.ds(j * L, L)]
            plsc.addupdate_scatter(hist, [iv], ones16)
            return carry

        lax.fori_loop(0, E_W // L, _body, 0)

    _accumulate(hist_o, src_hbm)
    _accumulate(hist_i, dst_hbm)

    pltpu.sync_copy(hist_o, hist_sh.at[sid, 0])
    pltpu.sync_copy(hist_i, hist_sh.at[sid, 1])
    plsc.subcore_barrier()
    pltpu.sync_copy(hist_sh.at[:, :, pl.ds(rbase, ROWS_W)], tmp_v)

    def _merge(g, carry):
        for d in range(2):
            acc = tmp_v[0, d, pl.ds(g * L, L)]
            for t in range(1, NS):
                acc = acc + tmp_v[t, d, pl.ds(g * L, L)]
            deg_v[d, pl.ds(g * L, L)] = acc
        return carry

    lax.fori_loop(0, ROWS_W // L, _merge, 0)

    def _norms(g, carry):
        do = deg_v[0, pl.ds(g * L, L)]
        ns_v[pl.ds(g * L, L)] = _rsqrt16(jnp.maximum(do, 1.0))
        di = deg_v[1, pl.ds(g * L, L)]
        deg_v[1, pl.ds(g * L, L)] = _rsqrt16(jnp.maximum(di, 1.0))
        return carry

    lax.fori_loop(0, ROWS_W // L, _norms, 0)
    pltpu.sync_copy(deg_v.at[1], nd_hbm.at[pl.ds(rbase, ROWS_W)])

    nk = ROWS_W // CH
    for b in range(NB_N):
        r0 = rbase + b * CH

        @pl.when(r0 < N)
        def _():
            pltpu.async_copy(fw_hbm.at[pl.ds(r0, CH)], row_v.at[b], fsems[b])

    for k in range(nk):
        r0 = rbase + k * CH
        b = k % NB_N

        @pl.when(r0 < N)
        def _():
            pltpu.make_async_copy(
                fw_hbm.at[pl.ds(0, CH)], row_v.at[b], fsems[b]).wait()

            def _scale(r, carry):
                s = plsc.load_gather(ns_v, [jnp.full((L,), k * CH + r, jnp.int32)])
                for j in range(F // L):
                    row_v[b, r, pl.ds(j * L, L)] = row_v[b, r, pl.ds(j * L, L)] * s
                return carry

            lax.fori_loop(0, CH, _scale, 0)
            pltpu.async_copy(row_v.at[b], hw_hbm.at[pl.ds(r0, CH)], wsems[b])
        if k + NB_N < nk:
            rn = rbase + (k + NB_N) * CH

            @pl.when(rn < N)
            def _():
                pltpu.make_async_copy(
                    fw_hbm.at[pl.ds(0, CH)], row_v.at[b], wsems[b]).wait()
                pltpu.async_copy(fw_hbm.at[pl.ds(rn, CH)], row_v.at[b],
                                 fsems[b])

    for b in range(NB_N):
        pltpu.make_async_copy(
            fw_hbm.at[pl.ds(0, CH)], row_v.at[b], wsems[b]).wait()


@functools.partial(
    pl.kernel,
    out_type=jax.ShapeDtypeStruct((N, F), jnp.float32),
    mesh=_sc_mesh,
    compiler_params=_sc_params,
    scratch_types=[
        pltpu.VMEM_SHARED((NP, F), jnp.float32),
        pltpu.VMEM((E_W,), jnp.int32),
        [pltpu.VMEM((CH,), jnp.int32)] * NBUF,
        pltpu.VMEM((NBUF, CH, F), jnp.float32),
        pltpu.VMEM((ZR, F), jnp.float32),
        pltpu.VMEM((ROWS_W,), jnp.float32),
        pltpu.VMEM((F,), jnp.float32),
        [pltpu.SemaphoreType.DMA] * NBUF,
        [pltpu.SemaphoreType.DMA] * NBUF,
        pltpu.SemaphoreType.DMA,
    ],
)
def _agg_kernel(src_hbm, dst_hbm, hw_hbm, nd_hbm, b_hbm, out_hbm, acc_sh, sidx_v, didxs,
                rows_v, zrow_v, nd_v, b_v, gsems, dsems, nsem):
    sid = lax.axis_index("s")
    ebase = sid * E_W
    rbase = sid * ROWS_W

    pltpu.sync_copy(src_hbm.at[pl.ds(ebase, E_W)], sidx_v)

    def _start_didx(c, b):
        pltpu.async_copy(dst_hbm.at[pl.ds(ebase + c * CH, CH)], didxs[b],
                         dsems[b])

    def _start_gather(c, b):
        pltpu.async_copy(
            hw_hbm.at[sidx_v.at[pl.ds(c * CH, CH)]], rows_v.at[b], gsems[b])

    for b in range(NBUF):
        _start_didx(b, b)
        _start_gather(b, b)

    pltpu.async_copy(nd_hbm.at[pl.ds(rbase, ROWS_W)], nd_v, nsem)
    pltpu.async_copy(b_hbm, b_v, nsem)

    zeros16 = jnp.zeros((L,), jnp.float32)

    def _zero_row(r, carry):
        for j in range(F // L):
            zrow_v[r, pl.ds(j * L, L)] = zeros16
        return carry

    lax.fori_loop(0, ZR, _zero_row, 0)
    for k in range(ROWS_W // ZR):
        pltpu.sync_copy(zrow_v, acc_sh.at[pl.ds(rbase + k * ZR, ZR)])
    pltpu.make_async_copy(nd_hbm.at[pl.ds(0, ROWS_W)], nd_v, nsem).wait()
    pltpu.make_async_copy(b_hbm, b_v, nsem).wait()
    plsc.subcore_barrier()

    def _outer(i, carry):
        c0 = i * NBUF
        for b in range(NBUF):
            c = c0 + b
            pltpu.make_async_copy(
                dst_hbm.at[pl.ds(0, CH)], didxs[b], dsems[b]).wait()
            pltpu.make_async_copy(
                hw_hbm.at[pl.ds(0, CH)], rows_v.at[b], gsems[b]).wait()
            pltpu.sync_copy(rows_v.at[b], acc_sh.at[didxs[b]], add=True)

            @pl.when(c + NBUF < NCHUNK)
            def _():
                _start_didx(c + NBUF, b)
                _start_gather(c + NBUF, b)

        return carry

    lax.fori_loop(0, NCHUNK // NBUF, _outer, 0)

    plsc.subcore_barrier()

    nk = ROWS_W // CH
    for b in range(NBUF):
        r0 = rbase + b * CH

        @pl.when(r0 < N)
        def _():
            pltpu.async_copy(acc_sh.at[pl.ds(r0, CH)], rows_v.at[b], gsems[b])

    for k in range(nk):
        r0 = rbase + k * CH
        b = k % NBUF

        @pl.when(r0 < N)
        def _():
            pltpu.make_async_copy(
                hw_hbm.at[pl.ds(0, CH)], rows_v.at[b], gsems[b]).wait()

            def _scale(r, carry):
                s = plsc.load_gather(nd_v, [jnp.full((L,), k * CH + r, jnp.int32)])
                for j in range(F // L):
                    rows_v[b, r, pl.ds(j * L, L)] = (
                        rows_v[b, r, pl.ds(j * L, L)] * s + b_v[pl.ds(j * L, L)])
                return carry

            lax.fori_loop(0, CH, _scale, 0)
            pltpu.async_copy(rows_v.at[b], out_hbm.at[pl.ds(r0, CH)], dsems[b])
        if k + NBUF < nk:
            rn = rbase + (k + NBUF) * CH

            @pl.when(rn < N)
            def _():
                pltpu.make_async_copy(
                    hw_hbm.at[pl.ds(0, CH)], rows_v.at[b], dsems[b]).wait()
                pltpu.async_copy(acc_sh.at[pl.ds(rn, CH)], rows_v.at[b],
                                 gsems[b])

    for b in range(NBUF):
        pltpu.make_async_copy(
            hw_hbm.at[pl.ds(0, CH)], rows_v.at[b], dsems[b]).wait()


def _mm_body(feat_ref, w_ref, fw_ref):
    fw_ref[...] = jnp.dot(feat_ref[...], w_ref[...],
                          preferred_element_type=jnp.float32)


def kernel(features, edge_index, W, b):
    fw = pl.pallas_call(
        _mm_body,
        grid=(N // RBLK,),
        in_specs=[
            pl.BlockSpec((RBLK, F), lambda i: (i, 0)),
            pl.BlockSpec((F, F), lambda i: (0, 0)),
        ],
        out_specs=pl.BlockSpec((RBLK, F), lambda i: (i, 0)),
        out_shape=jax.ShapeDtypeStruct((N, F), jnp.float32),
    )(features, W)

    src = edge_index[0]
    dst = edge_index[1]
    hw, nd = _norm_kernel(src, dst, fw)
    rst = _agg_kernel(src, dst, hw, nd, b)
    return (rst, features)


# speedup vs baseline: 1.3140x; 1.3140x over previous
import functools

import jax
import jax.numpy as jnp
from jax import lax
from jax.experimental import pallas as pl
from jax.experimental.pallas import tpu as pltpu
from jax.experimental.pallas import tpu_sc as plsc

N = 10000
E = 320000
F = 128

NS = 16
L = 16
NP = 10240
ROWS_W = NP // NS
E_W = E // NS
CH = 40
NCHUNK = E_W // CH
NBUF = 5
NB_N = 3
ZR = 16
RBLK = 1000

_sc_mesh = plsc.VectorSubcoreMesh(
    core_axis_name="c", subcore_axis_name="s", num_cores=1)
_sc_params = pltpu.CompilerParams(needs_layout_passes=False)


def _rsqrt16(x):
    i = plsc.bitcast(x, jnp.int32)
    y = plsc.bitcast(jnp.int32(0x5F3759DF) - (i >> 1), jnp.float32)
    half = x * 0.5
    for _ in range(3):
        y = y * (1.5 - half * y * y)
    return y


@functools.partial(
    pl.kernel,
    out_type=(
        jax.ShapeDtypeStruct((N, F), jnp.float32),
        jax.ShapeDtypeStruct((NP,), jnp.float32),
    ),
    mesh=_sc_mesh,
    compiler_params=_sc_params,
    scratch_types=[
        pltpu.VMEM_SHARED((NS, 2, NP), jnp.float32),
        pltpu.VMEM((E_W,), jnp.int32),
        pltpu.VMEM((NP,), jnp.float32),
        pltpu.VMEM((NP,), jnp.float32),
        pltpu.VMEM((NS, 2, ROWS_W), jnp.float32),
        pltpu.VMEM((2, ROWS_W), jnp.float32),
        pltpu.VMEM((ROWS_W,), jnp.float32),
        pltpu.VMEM((NB_N, CH, F), jnp.float32),
        [pltpu.SemaphoreType.DMA] * NB_N,
        [pltpu.SemaphoreType.DMA] * NB_N,
    ],
)
def _norm_kernel(src_hbm, dst_hbm, fw_hbm, hw_hbm, nd_hbm, hist_sh, eidx_v, hist_o,
                 hist_i, tmp_v, deg_v, ns_v, row_v, fsems, wsems):
    sid = lax.axis_index("s")
    base = sid * E_W
    rbase = sid * ROWS_W

    zeros16 = jnp.zeros((L,), jnp.float32)

    def _zero(i, carry):
        hist_o[pl.ds(i * L, L)] = zeros16
        hist_i[pl.ds(i * L, L)] = zeros16
        return carry

    lax.fori_loop(0, NP // L, _zero, 0)

    ones16 = jnp.ones((L,), jnp.float32)

    def _accumulate(hist, ehbm):
        pltpu.sync_copy(ehbm.at[pl.ds(base, E_W)], eidx_v)

        def _body(j, carry):
            iv = eidx_v[pl.ds(j * L, L)]
            plsc.addupdate_scatter(hist, [iv], ones16)
            return carry

        lax.fori_loop(0, E_W // L, _body, 0)

    _accumulate(hist_o, src_hbm)
    _accumulate(hist_i, dst_hbm)

    pltpu.sync_copy(hist_o, hist_sh.at[sid, 0])
    pltpu.sync_copy(hist_i, hist_sh.at[sid, 1])
    plsc.subcore_barrier()
    pltpu.sync_copy(hist_sh.at[:, :, pl.ds(rbase, ROWS_W)], tmp_v)

    def _merge(g, carry):
        for d in range(2):
            acc = tmp_v[0, d, pl.ds(g * L, L)]
            for t in range(1, NS):
                acc = acc + tmp_v[t, d, pl.ds(g * L, L)]
            deg_v[d, pl.ds(g * L, L)] = acc
        return carry

    lax.fori_loop(0, ROWS_W // L, _merge, 0)

    def _norms(g, carry):
        do = deg_v[0, pl.ds(g * L, L)]
        ns_v[pl.ds(g * L, L)] = _rsqrt16(jnp.maximum(do, 1.0))
        di = deg_v[1, pl.ds(g * L, L)]
        deg_v[1, pl.ds(g * L, L)] = _rsqrt16(jnp.maximum(di, 1.0))
        return carry

    lax.fori_loop(0, ROWS_W // L, _norms, 0)
    pltpu.sync_copy(deg_v.at[1], nd_hbm.at[pl.ds(rbase, ROWS_W)])

    nk = ROWS_W // CH
    for b in range(NB_N):
        r0 = rbase + b * CH

        @pl.when(r0 < N)
        def _():
            pltpu.async_copy(fw_hbm.at[pl.ds(r0, CH)], row_v.at[b], fsems[b])

    for k in range(nk):
        r0 = rbase + k * CH
        b = k % NB_N

        @pl.when(r0 < N)
        def _():
            pltpu.make_async_copy(
                fw_hbm.at[pl.ds(0, CH)], row_v.at[b], fsems[b]).wait()

            def _scale(r, carry):
                s = plsc.load_gather(ns_v, [jnp.full((L,), k * CH + r, jnp.int32)])
                for j in range(F // L):
                    row_v[b, r, pl.ds(j * L, L)] = row_v[b, r, pl.ds(j * L, L)] * s
                return carry

            lax.fori_loop(0, CH, _scale, 0)
            pltpu.async_copy(row_v.at[b], hw_hbm.at[pl.ds(r0, CH)], wsems[b])
        if k + NB_N < nk:
            rn = rbase + (k + NB_N) * CH

            @pl.when(rn < N)
            def _():
                pltpu.make_async_copy(
                    fw_hbm.at[pl.ds(0, CH)], row_v.at[b], wsems[b]).wait()
                pltpu.async_copy(fw_hbm.at[pl.ds(rn, CH)], row_v.at[b],
                                 fsems[b])

    for b in range(NB_N):
        pltpu.make_async_copy(
            fw_hbm.at[pl.ds(0, CH)], row_v.at[b], wsems[b]).wait()


@functools.partial(
    pl.kernel,
    out_type=jax.ShapeDtypeStruct((N, F), jnp.float32),
    mesh=_sc_mesh,
    compiler_params=_sc_params,
    scratch_types=[
        pltpu.VMEM_SHARED((NP, F), jnp.float32),
        pltpu.VMEM((E_W,), jnp.int32),
        [pltpu.VMEM((CH,), jnp.int32)] * NBUF,
        pltpu.VMEM((NBUF, CH, F), jnp.float32),
        pltpu.VMEM((ZR, F), jnp.float32),
        pltpu.VMEM((ROWS_W,), jnp.float32),
        pltpu.VMEM((F,), jnp.float32),
        [pltpu.SemaphoreType.DMA] * NBUF,
        [pltpu.SemaphoreType.DMA] * NBUF,
        pltpu.SemaphoreType.DMA,
    ],
)
def _agg_kernel(src_hbm, dst_hbm, hw_hbm, nd_hbm, b_hbm, out_hbm, acc_sh, sidx_v, didxs,
                rows_v, zrow_v, nd_v, b_v, gsems, dsems, nsem):
    sid = lax.axis_index("s")
    ebase = sid * E_W
    rbase = sid * ROWS_W

    pltpu.sync_copy(src_hbm.at[pl.ds(ebase, E_W)], sidx_v)

    def _start_didx(c, b):
        pltpu.async_copy(dst_hbm.at[pl.ds(ebase + c * CH, CH)], didxs[b],
                         dsems[b])

    def _start_gather(c, b):
        pltpu.async_copy(
            hw_hbm.at[sidx_v.at[pl.ds(c * CH, CH)]], rows_v.at[b], gsems[b])

    for b in range(NBUF):
        _start_didx(b, b)
        _start_gather(b, b)

    pltpu.async_copy(nd_hbm.at[pl.ds(rbase, ROWS_W)], nd_v, nsem)
    pltpu.async_copy(b_hbm, b_v, nsem)

    zeros16 = jnp.zeros((L,), jnp.float32)

    def _zero_row(r, carry):
        for j in range(F // L):
            zrow_v[r, pl.ds(j * L, L)] = zeros16
        return carry

    lax.fori_loop(0, ZR, _zero_row, 0)
    for k in range(ROWS_W // ZR):
        pltpu.sync_copy(zrow_v, acc_sh.at[pl.ds(rbase + k * ZR, ZR)])
    pltpu.make_async_copy(nd_hbm.at[pl.ds(0, ROWS_W)], nd_v, nsem).wait()
    pltpu.make_async_copy(b_hbm, b_v, nsem).wait()
    plsc.subcore_barrier()

    def _outer(i, carry):
        c0 = i * NBUF
        for b in range(NBUF):
            c = c0 + b
            pltpu.make_async_copy(
                dst_hbm.at[pl.ds(0, CH)], didxs[b], dsems[b]).wait()
            pltpu.make_async_copy(
                hw_hbm.at[pl.ds(0, CH)], rows_v.at[b], gsems[b]).wait()
            pltpu.sync_copy(rows_v.at[b], acc_sh.at[didxs[b]], add=True)

            @pl.when(c + NBUF < NCHUNK)
            def _():
                _start_didx(c + NBUF, b)
                _start_gather(c + NBUF, b)

        return carry

    lax.fori_loop(0, NCHUNK // NBUF, _outer, 0)

    plsc.subcore_barrier()

    nk = ROWS_W // CH
    for b in range(NBUF):
        r0 = rbase + b * CH

        @pl.when(r0 < N)
        def _():
            pltpu.async_copy(acc_sh.at[pl.ds(r0, CH)], rows_v.at[b], gsems[b])

    for k in range(nk):
        r0 = rbase + k * CH
        b = k % NBUF

        @pl.when(r0 < N)
        def _():
            pltpu.make_async_copy(
                hw_hbm.at[pl.ds(0, CH)], rows_v.at[b], gsems[b]).wait()

            def _scale(r, carry):
                s = plsc.load_gather(nd_v, [jnp.full((L,), k * CH + r, jnp.int32)])
                for j in range(F // L):
                    rows_v[b, r, pl.ds(j * L, L)] = (
                        rows_v[b, r, pl.ds(j * L, L)] * s + b_v[pl.ds(j * L, L)])
                return carry

            lax.fori_loop(0, CH, _scale, 0)
            pltpu.async_copy(rows_v.at[b], out_hbm.at[pl.ds(r0, CH)], dsems[b])
        if k + NBUF < nk:
            rn = rbase + (k + NBUF) * CH

            @pl.when(rn < N)
            def _():
                pltpu.make_async_copy(
                    hw_hbm.at[pl.ds(0, CH)], rows_v.at[b], dsems[b]).wait()
                pltpu.async_copy(acc_sh.at[pl.ds(rn, CH)], rows_v.at[b],
                                 gsems[b])

    for b in range(NBUF):
        pltpu.make_async_copy(
            hw_hbm.at[pl.ds(0, CH)], rows_v.at[b], dsems[b]).wait()


def _mm_body(feat_ref, w_ref, fw_ref):
    fw_ref[...] = jnp.dot(feat_ref[...], w_ref[...],
                          preferred_element_type=jnp.float32)


def kernel(features, edge_index, W, b):
    fw = pl.pallas_call(
        _mm_body,
        grid=(N // RBLK,),
        in_specs=[
            pl.BlockSpec((RBLK, F), lambda i: (i, 0)),
            pl.BlockSpec((F, F), lambda i: (0, 0)),
        ],
        out_specs=pl.BlockSpec((RBLK, F), lambda i: (i, 0)),
        out_shape=jax.ShapeDtypeStruct((N, F), jnp.float32),
    )(features, W)

    src = edge_index[0]
    dst = edge_index[1]
    hw, nd = _norm_kernel(src, dst, fw)
    rst = _agg_kernel(src, dst, hw, nd, b)
    return (rst, features)
